# baseline (speedup 1.0000x reference)
import functools
import jax
import jax.numpy as jnp
from jax import lax
from jax.experimental import pallas as pl
from jax.experimental.pallas import tpu as pltpu

N = 10000
E = 160000
DIN = 128
H = 32
ED = 16
EH = 128
STEPS = 3
EPS = 1e-5

_INTERPRET = False


def _lin0_body(x_ref, w_ref, b_ref, o_ref):
    y = jnp.dot(x_ref[...], w_ref[...], preferred_element_type=jnp.float32)
    o_ref[...] = jnp.maximum(y + b_ref[...], 0.0)


def _lin0(n_feat, lin0_W, lin0_b):
    return pl.pallas_call(
        _lin0_body,
        out_shape=jax.ShapeDtypeStruct((N, H), jnp.float32),
        interpret=_INTERPRET,
    )(n_feat, lin0_W, lin0_b.reshape(1, H))


_BE_W = 640


def _wnet_body(ef_ref, w1_ref, b1_ref, w2_ref, b2_ref, o_ref):
    eh = jnp.dot(ef_ref[...], w1_ref[...], preferred_element_type=jnp.float32)
    eh = jnp.maximum(eh + b1_ref[...], 0.0)
    o_ref[...] = jnp.dot(eh, w2_ref[...], preferred_element_type=jnp.float32) + b2_ref[...]


def _wnet(e_feat, en1_W, en1_b, en2_W, en2_b):
    grid = (E // _BE_W,)
    return pl.pallas_call(
        _wnet_body,
        grid=grid,
        in_specs=[
            pl.BlockSpec((_BE_W, ED), lambda i: (i, 0)),
            pl.BlockSpec((ED, EH), lambda i: (0, 0)),
            pl.BlockSpec((1, EH), lambda i: (0, 0)),
            pl.BlockSpec((EH, H * H), lambda i: (0, 0)),
            pl.BlockSpec((1, H * H), lambda i: (0, 0)),
        ],
        out_specs=pl.BlockSpec((_BE_W, H * H), lambda i: (i, 0)),
        out_shape=jax.ShapeDtypeStruct((E, H * H), jnp.float32),
        interpret=_INTERPRET,
    )(e_feat, en1_W, en1_b.reshape(1, EH), en2_W, en2_b.reshape(1, H * H))


_BE_M = 640


def _msg_body(s_ref, w_ref, o_ref):
    s = s_ref[...]
    w = w_ref[...].reshape(_BE_M, H, H)
    o_ref[...] = jnp.sum(s[:, :, None] * w, axis=1)


def _msg(s, w):
    grid = (E // _BE_M,)
    return pl.pallas_call(
        _msg_body,
        grid=grid,
        in_specs=[
            pl.BlockSpec((_BE_M, H), lambda i: (i, 0)),
            pl.BlockSpec((_BE_M, H * H), lambda i: (i, 0)),
        ],
        out_specs=pl.BlockSpec((_BE_M, H), lambda i: (i, 0)),
        out_shape=jax.ShapeDtypeStruct((E, H), jnp.float32),
        interpret=_INTERPRET,
    )(s, w)


def _final_body(a_ref, g_ref, bt_ref, w1_ref, b1_ref, w2_ref, b2_ref,
                y1_ref, y2_ref):
    out = jnp.maximum(a_ref[...], 0.0)
    mean = jnp.mean(out, axis=0, keepdims=True)
    var = jnp.mean((out - mean) ** 2, axis=0, keepdims=True)
    yb = (out - mean) * lax.rsqrt(var + EPS) * g_ref[...] + bt_ref[...]
    z1 = jnp.dot(yb, w1_ref[...], preferred_element_type=jnp.float32) + b1_ref[...]
    z2 = jnp.dot(yb, w2_ref[...], preferred_element_type=jnp.float32) + b2_ref[...]
    y1_ref[...] = jax.nn.sigmoid(z1)
    y2_ref[...] = jax.nn.sigmoid(z2)


def _final(agg, bn_gamma, bn_beta, yl_W, yl_b, yl2_W, yl2_b):
    return pl.pallas_call(
        _final_body,
        out_shape=(
            jax.ShapeDtypeStruct((N, 2), jnp.float32),
            jax.ShapeDtypeStruct((N, 1), jnp.float32),
        ),
        interpret=_INTERPRET,
    )(agg, bn_gamma.reshape(1, H), bn_beta.reshape(1, H),
      yl_W, yl_b.reshape(1, 2), yl2_W, yl2_b.reshape(1, 1))


def kernel(n_feat, edge_index, e_feat, lin0_W, lin0_b, en1_W, en1_b,
           en2_W, en2_b, bn_gamma, bn_beta, yl_W, yl_b, yl2_W, yl2_b):
    src = edge_index[0].astype(jnp.int32)
    dst = edge_index[1].astype(jnp.int32)

    out = _lin0(n_feat, lin0_W, lin0_b)
    w = _wnet(e_feat, en1_W, en1_b, en2_W, en2_b)

    for step in range(STEPS):
        s = out[src]
        msg = _msg(s, w)
        agg = jax.ops.segment_sum(msg, dst, num_segments=N)
        if step < STEPS - 1:
            out = jnp.maximum(agg, 0.0)

    return _final(agg, bn_gamma, bn_beta, yl_W, yl_b, yl2_W, yl2_b)


# speedup vs baseline: 1.2313x; 1.2313x over previous
import functools
import jax
import jax.numpy as jnp
from jax import lax
from jax.experimental import pallas as pl
from jax.experimental.pallas import tpu as pltpu
from jax.experimental.pallas import tpu_sc as plsc

N = 10000
E = 160000
DIN = 128
H = 32
ED = 16
EH = 128
STEPS = 3
EPS = 1e-5

_INTERPRET = False

_NC = 2
_NS = 16
_NW = _NC * _NS
_T = 128
_KT = 40
_EPW = _T * _KT
_EPAD = _NW * _EPW
_CH = 8
_NCHUNK = _KT // _CH
_NPT = N // _NS


def _lin0_body(x_ref, w_ref, b_ref, o_ref):
    y = jnp.dot(x_ref[...], w_ref[...], preferred_element_type=jnp.float32)
    o_ref[...] = jnp.maximum(y + b_ref[...], 0.0)


def _lin0(n_feat, lin0_W, lin0_b):
    return pl.pallas_call(
        _lin0_body,
        out_shape=jax.ShapeDtypeStruct((N, H), jnp.float32),
        interpret=_INTERPRET,
    )(n_feat, lin0_W, lin0_b.reshape(1, H))


_BE_W = 640


def _wnet_body(ef_ref, w1_ref, b1_ref, w2_ref, b2_ref, o_ref):
    eh = jnp.dot(ef_ref[...], w1_ref[...], preferred_element_type=jnp.float32)
    eh = jnp.maximum(eh + b1_ref[...], 0.0)
    o_ref[...] = jnp.dot(eh, w2_ref[...], preferred_element_type=jnp.float32) + b2_ref[...]


def _wnet(e_feat, en1_W, en1_b, en2_W, en2_b):
    grid = (E // _BE_W,)
    return pl.pallas_call(
        _wnet_body,
        grid=grid,
        in_specs=[
            pl.BlockSpec((_BE_W, ED), lambda i: (i, 0)),
            pl.BlockSpec((ED, EH), lambda i: (0, 0)),
            pl.BlockSpec((1, EH), lambda i: (0, 0)),
            pl.BlockSpec((EH, H * H), lambda i: (0, 0)),
            pl.BlockSpec((1, H * H), lambda i: (0, 0)),
        ],
        out_specs=pl.BlockSpec((_BE_W, H * H), lambda i: (i, 0)),
        out_shape=jax.ShapeDtypeStruct((E, H * H), jnp.float32),
        interpret=_INTERPRET,
    )(e_feat, en1_W, en1_b.reshape(1, EH), en2_W, en2_b.reshape(1, H * H))


_BE_M = 640
_NT_REAL = E // _BE_M
_NT_PAD = _EPAD // _BE_M


def _msg_body_single(s_ref, w_ref, o_ref):
    i = pl.program_id(0)
    s = s_ref[...]
    w = w_ref[...].reshape(_BE_M, H, H)
    m = jnp.sum(s[:, :, None] * w, axis=1)
    o_ref[...] = m * jnp.where(i < _NT_REAL, 1.0, 0.0)


def _msg_body_pair(s0_ref, s1_ref, w_ref, o_ref):
    i = pl.program_id(0)
    s = jnp.maximum(s0_ref[...] + s1_ref[...], 0.0)
    w = w_ref[...].reshape(_BE_M, H, H)
    m = jnp.sum(s[:, :, None] * w, axis=1)
    o_ref[...] = m * jnp.where(i < _NT_REAL, 1.0, 0.0)


def _w_index(i):
    return (jnp.minimum(i, _NT_REAL - 1), 0)


def _msg_single(s, w):
    return pl.pallas_call(
        _msg_body_single,
        grid=(_NT_PAD,),
        in_specs=[
            pl.BlockSpec((_BE_M, H), lambda i: (i, 0)),
            pl.BlockSpec((_BE_M, H * H), _w_index),
        ],
        out_specs=pl.BlockSpec((_BE_M, H), lambda i: (i, 0)),
        out_shape=jax.ShapeDtypeStruct((_EPAD, H), jnp.float32),
        interpret=_INTERPRET,
    )(s, w)


def _msg_pair(s0, s1, w):
    return pl.pallas_call(
        _msg_body_pair,
        grid=(_NT_PAD,),
        in_specs=[
            pl.BlockSpec((_BE_M, H), lambda i: (i, 0)),
            pl.BlockSpec((_BE_M, H), lambda i: (i, 0)),
            pl.BlockSpec((_BE_M, H * H), _w_index),
        ],
        out_specs=pl.BlockSpec((_BE_M, H), lambda i: (i, 0)),
        out_shape=jax.ShapeDtypeStruct((_EPAD, H), jnp.float32),
        interpret=_INTERPRET,
    )(s0, s1, w)


@functools.lru_cache(maxsize=None)
def _build_gather1():
    mesh = plsc.VectorSubcoreMesh(core_axis_name="c", subcore_axis_name="s")

    @functools.partial(
        pl.kernel,
        out_type=jax.ShapeDtypeStruct((_EPAD, H), jnp.float32),
        mesh=mesh,
        compiler_params=pltpu.CompilerParams(use_tc_tiling_on_sc=False),
        scratch_types=[
            pltpu.VMEM((_KT, _T), jnp.int32),
            pltpu.VMEM((_CH * _T, H), jnp.float32),
            pltpu.SemaphoreType.DMA,
        ],
    )
    def gather1(tab_hbm, idx_hbm, s_hbm, idx_v, rows_v, sem):
        wid = lax.axis_index("s") * _NC + lax.axis_index("c")
        pltpu.sync_copy(idx_hbm.at[pl.ds(wid * _KT, _KT)], idx_v)
        for c in range(_NCHUNK):
            cps = []
            for b in range(_CH):
                cps.append(pltpu.async_copy(
                    tab_hbm.at[idx_v.at[c * _CH + b]],
                    rows_v.at[pl.ds(b * _T, _T)], sem))
            for cp in cps:
                cp.wait()
            pltpu.sync_copy(
                rows_v, s_hbm.at[pl.ds(wid * _EPW + c * _CH * _T, _CH * _T)])

    return gather1


@functools.lru_cache(maxsize=None)
def _build_gather2():
    mesh = plsc.VectorSubcoreMesh(core_axis_name="c", subcore_axis_name="s")

    @functools.partial(
        pl.kernel,
        out_type=(jax.ShapeDtypeStruct((_EPAD, H), jnp.float32),
                  jax.ShapeDtypeStruct((_EPAD, H), jnp.float32)),
        mesh=mesh,
        compiler_params=pltpu.CompilerParams(use_tc_tiling_on_sc=False),
        scratch_types=[
            pltpu.VMEM((_KT, _T), jnp.int32),
            pltpu.VMEM((_CH * _T, H), jnp.float32),
            pltpu.VMEM((_CH * _T, H), jnp.float32),
            pltpu.SemaphoreType.DMA,
        ],
    )
    def gather2(t0_hbm, t1_hbm, idx_hbm, s0_hbm, s1_hbm, idx_v, r0_v, r1_v, sem):
        wid = lax.axis_index("s") * _NC + lax.axis_index("c")
        pltpu.sync_copy(idx_hbm.at[pl.ds(wid * _KT, _KT)], idx_v)
        for c in range(_NCHUNK):
            cps = []
            for b in range(_CH):
                iv = idx_v.at[c * _CH + b]
                cps.append(pltpu.async_copy(
                    t0_hbm.at[iv], r0_v.at[pl.ds(b * _T, _T)], sem))
                cps.append(pltpu.async_copy(
                    t1_hbm.at[iv], r1_v.at[pl.ds(b * _T, _T)], sem))
            for cp in cps:
                cp.wait()
            dst = pl.ds(wid * _EPW + c * _CH * _T, _CH * _T)
            pltpu.sync_copy(r0_v, s0_hbm.at[dst])
            pltpu.sync_copy(r1_v, s1_hbm.at[dst])

    return gather2


@functools.lru_cache(maxsize=None)
def _build_scatter():
    mesh = plsc.VectorSubcoreMesh(core_axis_name="c", subcore_axis_name="s")

    @functools.partial(
        pl.kernel,
        out_type=jax.ShapeDtypeStruct((_NC, N, H), jnp.float32),
        mesh=mesh,
        compiler_params=pltpu.CompilerParams(use_tc_tiling_on_sc=False),
        scratch_types=[
            pltpu.VMEM((_KT, _T), jnp.int32),
            pltpu.VMEM((_CH * _T, H), jnp.float32),
            pltpu.VMEM((_NPT, H), jnp.float32),
            pltpu.VMEM_SHARED((N, H), jnp.float32),
            pltpu.SemaphoreType.DMA,
        ],
    )
    def scatter(msg_hbm, idx_hbm, zer_hbm, p_hbm, idx_v, msg_v, nbuf, acc_sh, sem):
        cid = lax.axis_index("c")
        sid = lax.axis_index("s")
        wid = sid * _NC + cid
        nrows = pl.ds(sid * _NPT, _NPT)
        pltpu.sync_copy(zer_hbm.at[nrows], nbuf)
        pltpu.sync_copy(nbuf, acc_sh.at[nrows])
        plsc.subcore_barrier()
        pltpu.sync_copy(idx_hbm.at[pl.ds(wid * _KT, _KT)], idx_v)
        for c in range(_NCHUNK):
            pltpu.sync_copy(
                msg_hbm.at[pl.ds(wid * _EPW + c * _CH * _T, _CH * _T)], msg_v)
            for b in range(_CH):
                pltpu.sync_copy(msg_v.at[pl.ds(b * _T, _T)],
                                acc_sh.at[idx_v.at[c * _CH + b]], add=True)
        plsc.subcore_barrier()
        pltpu.sync_copy(acc_sh.at[nrows], nbuf)
        pltpu.sync_copy(nbuf, p_hbm.at[cid, nrows])

    return scatter


def _final_body(p_ref, g_ref, bt_ref, w1_ref, b1_ref, w2_ref, b2_ref,
                y1_ref, y2_ref):
    p = p_ref[...]
    out = jnp.maximum(p[0] + p[1], 0.0)
    mean = jnp.mean(out, axis=0, keepdims=True)
    var = jnp.mean((out - mean) ** 2, axis=0, keepdims=True)
    yb = (out - mean) * lax.rsqrt(var + EPS) * g_ref[...] + bt_ref[...]
    z1 = jnp.dot(yb, w1_ref[...], preferred_element_type=jnp.float32) + b1_ref[...]
    z2 = jnp.dot(yb, w2_ref[...], preferred_element_type=jnp.float32) + b2_ref[...]
    y1_ref[...] = jax.nn.sigmoid(z1)
    y2_ref[...] = jax.nn.sigmoid(z2)


def _final(p, bn_gamma, bn_beta, yl_W, yl_b, yl2_W, yl2_b):
    return pl.pallas_call(
        _final_body,
        out_shape=(
            jax.ShapeDtypeStruct((N, 2), jnp.float32),
            jax.ShapeDtypeStruct((N, 1), jnp.float32),
        ),
        interpret=_INTERPRET,
    )(p, bn_gamma.reshape(1, H), bn_beta.reshape(1, H),
      yl_W, yl_b.reshape(1, 2), yl2_W, yl2_b.reshape(1, 1))


def kernel(n_feat, edge_index, e_feat, lin0_W, lin0_b, en1_W, en1_b,
           en2_W, en2_b, bn_gamma, bn_beta, yl_W, yl_b, yl2_W, yl2_b):
    src = edge_index[0].astype(jnp.int32)
    dst = edge_index[1].astype(jnp.int32)
    pad = jnp.zeros((_EPAD - E,), jnp.int32)
    src2d = jnp.concatenate([src, pad]).reshape(_EPAD // _T, _T)
    dst2d = jnp.concatenate([dst, pad]).reshape(_EPAD // _T, _T)
    zer = jnp.zeros((N, H), jnp.float32)

    out = _lin0(n_feat, lin0_W, lin0_b)
    w = _wnet(e_feat, en1_W, en1_b, en2_W, en2_b)

    gather1 = _build_gather1()
    gather2 = _build_gather2()
    scatter = _build_scatter()

    s = gather1(out, src2d)
    msg = _msg_single(s, w)
    p = scatter(msg, dst2d, zer)
    for _ in range(STEPS - 1):
        s0, s1 = gather2(p[0], p[1], src2d)
        msg = _msg_pair(s0, s1, w)
        p = scatter(msg, dst2d, zer)

    return _final(p, bn_gamma, bn_beta, yl_W, yl_b, yl2_W, yl2_b)


# speedup vs baseline: 2.6702x; 2.1685x over previous
import functools
import jax
import jax.numpy as jnp
from jax import lax
from jax.experimental import pallas as pl
from jax.experimental.pallas import tpu as pltpu
from jax.experimental.pallas import tpu_sc as plsc

N = 10000
E = 160000
DIN = 128
H = 32
ED = 16
EH = 128
STEPS = 3
EPS = 1e-5

_INTERPRET = False

_NC = 2
_NS = 16
_NW = _NC * _NS
_T = 128
_KT = 40
_EPW = _T * _KT
_EPAD = _NW * _EPW
_CH = 8
_NCHUNK = _KT // _CH
_NPT = N // _NS


def _lin0_body(x_ref, w_ref, b_ref, o_ref):
    y = jnp.dot(x_ref[...], w_ref[...], preferred_element_type=jnp.float32)
    o_ref[...] = jnp.maximum(y + b_ref[...], 0.0)


def _lin0(n_feat, lin0_W, lin0_b):
    return pl.pallas_call(
        _lin0_body,
        out_shape=jax.ShapeDtypeStruct((N, H), jnp.float32),
        interpret=_INTERPRET,
    )(n_feat, lin0_W, lin0_b.reshape(1, H))


_BE_W = 640


def _wnet_body(ef_ref, w1_ref, b1_ref, w2_ref, b2_ref, o_ref):
    eh = jnp.dot(ef_ref[...], w1_ref[...], preferred_element_type=jnp.float32)
    eh = jnp.maximum(eh + b1_ref[...], 0.0)
    o_ref[...] = jnp.dot(eh, w2_ref[...], preferred_element_type=jnp.float32) + b2_ref[...]


def _wnet(e_feat, en1_W, en1_b, en2_W, en2_b):
    grid = (E // _BE_W,)
    return pl.pallas_call(
        _wnet_body,
        grid=grid,
        in_specs=[
            pl.BlockSpec((_BE_W, ED), lambda i: (i, 0)),
            pl.BlockSpec((ED, EH), lambda i: (0, 0)),
            pl.BlockSpec((1, EH), lambda i: (0, 0)),
            pl.BlockSpec((EH, H * H), lambda i: (0, 0)),
            pl.BlockSpec((1, H * H), lambda i: (0, 0)),
        ],
        out_specs=pl.BlockSpec((_BE_W, H * H), lambda i: (i, 0)),
        out_shape=jax.ShapeDtypeStruct((E, H * H), jnp.float32),
        interpret=_INTERPRET,
    )(e_feat, en1_W, en1_b.reshape(1, EH), en2_W, en2_b.reshape(1, H * H))


_BE_M = 640
_NT_REAL = E // _BE_M
_NT_PAD = _EPAD // _BE_M


def _msg_compute(s, w_ref, q_ref, r_ref, o_ref, i):
    srep = jnp.dot(s, q_ref[...], preferred_element_type=jnp.float32)
    m = jnp.dot(srep * w_ref[...], r_ref[...], preferred_element_type=jnp.float32)
    o_ref[...] = m * jnp.where(i < _NT_REAL, 1.0, 0.0)


def _msg_body_single(s_ref, w_ref, q_ref, r_ref, o_ref):
    _msg_compute(s_ref[...], w_ref, q_ref, r_ref, o_ref, pl.program_id(0))


def _msg_body_pair(s0_ref, s1_ref, w_ref, q_ref, r_ref, o_ref):
    s = jnp.maximum(s0_ref[...] + s1_ref[...], 0.0)
    _msg_compute(s, w_ref, q_ref, r_ref, o_ref, pl.program_id(0))


def _make_qr():
    j = jnp.arange(H * H)
    q = (j[None, :] // H == jnp.arange(H)[:, None]).astype(jnp.float32)
    r = (j[:, None] % H == jnp.arange(H)[None, :]).astype(jnp.float32)
    return q, r


def _w_index(i):
    return (jnp.minimum(i, _NT_REAL - 1), 0)


_QR_SPECS = [
    pl.BlockSpec((H, H * H), lambda i: (0, 0)),
    pl.BlockSpec((H * H, H), lambda i: (0, 0)),
]


def _msg_single(s, w, q, r):
    return pl.pallas_call(
        _msg_body_single,
        grid=(_NT_PAD,),
        in_specs=[
            pl.BlockSpec((_BE_M, H), lambda i: (i, 0)),
            pl.BlockSpec((_BE_M, H * H), _w_index),
        ] + _QR_SPECS,
        out_specs=pl.BlockSpec((_BE_M, H), lambda i: (i, 0)),
        out_shape=jax.ShapeDtypeStruct((_EPAD, H), jnp.float32),
        interpret=_INTERPRET,
    )(s, w, q, r)


def _msg_pair(s0, s1, w, q, r):
    return pl.pallas_call(
        _msg_body_pair,
        grid=(_NT_PAD,),
        in_specs=[
            pl.BlockSpec((_BE_M, H), lambda i: (i, 0)),
            pl.BlockSpec((_BE_M, H), lambda i: (i, 0)),
            pl.BlockSpec((_BE_M, H * H), _w_index),
        ] + _QR_SPECS,
        out_specs=pl.BlockSpec((_BE_M, H), lambda i: (i, 0)),
        out_shape=jax.ShapeDtypeStruct((_EPAD, H), jnp.float32),
        interpret=_INTERPRET,
    )(s0, s1, w, q, r)


@functools.lru_cache(maxsize=None)
def _build_gather1():
    mesh = plsc.VectorSubcoreMesh(core_axis_name="c", subcore_axis_name="s")

    @functools.partial(
        pl.kernel,
        out_type=jax.ShapeDtypeStruct((_EPAD, H), jnp.float32),
        mesh=mesh,
        compiler_params=pltpu.CompilerParams(use_tc_tiling_on_sc=False),
        scratch_types=[
            pltpu.VMEM((_KT, _T), jnp.int32),
            pltpu.VMEM((_CH * _T, H), jnp.float32),
            pltpu.SemaphoreType.DMA,
        ],
    )
    def gather1(tab_hbm, idx_hbm, s_hbm, idx_v, rows_v, sem):
        wid = lax.axis_index("s") * _NC + lax.axis_index("c")
        pltpu.sync_copy(idx_hbm.at[pl.ds(wid * _KT, _KT)], idx_v)
        for c in range(_NCHUNK):
            cps = []
            for b in range(_CH):
                cps.append(pltpu.async_copy(
                    tab_hbm.at[idx_v.at[c * _CH + b]],
                    rows_v.at[pl.ds(b * _T, _T)], sem))
            for cp in cps:
                cp.wait()
            pltpu.sync_copy(
                rows_v, s_hbm.at[pl.ds(wid * _EPW + c * _CH * _T, _CH * _T)])

    return gather1


@functools.lru_cache(maxsize=None)
def _build_gather2():
    mesh = plsc.VectorSubcoreMesh(core_axis_name="c", subcore_axis_name="s")

    @functools.partial(
        pl.kernel,
        out_type=(jax.ShapeDtypeStruct((_EPAD, H), jnp.float32),
                  jax.ShapeDtypeStruct((_EPAD, H), jnp.float32)),
        mesh=mesh,
        compiler_params=pltpu.CompilerParams(use_tc_tiling_on_sc=False),
        scratch_types=[
            pltpu.VMEM((_KT, _T), jnp.int32),
            pltpu.VMEM((_CH * _T, H), jnp.float32),
            pltpu.VMEM((_CH * _T, H), jnp.float32),
            pltpu.SemaphoreType.DMA,
        ],
    )
    def gather2(t0_hbm, t1_hbm, idx_hbm, s0_hbm, s1_hbm, idx_v, r0_v, r1_v, sem):
        wid = lax.axis_index("s") * _NC + lax.axis_index("c")
        pltpu.sync_copy(idx_hbm.at[pl.ds(wid * _KT, _KT)], idx_v)
        for c in range(_NCHUNK):
            cps = []
            for b in range(_CH):
                iv = idx_v.at[c * _CH + b]
                cps.append(pltpu.async_copy(
                    t0_hbm.at[iv], r0_v.at[pl.ds(b * _T, _T)], sem))
                cps.append(pltpu.async_copy(
                    t1_hbm.at[iv], r1_v.at[pl.ds(b * _T, _T)], sem))
            for cp in cps:
                cp.wait()
            dst = pl.ds(wid * _EPW + c * _CH * _T, _CH * _T)
            pltpu.sync_copy(r0_v, s0_hbm.at[dst])
            pltpu.sync_copy(r1_v, s1_hbm.at[dst])

    return gather2


@functools.lru_cache(maxsize=None)
def _build_scatter():
    mesh = plsc.VectorSubcoreMesh(core_axis_name="c", subcore_axis_name="s")

    @functools.partial(
        pl.kernel,
        out_type=jax.ShapeDtypeStruct((_NC, N, H), jnp.float32),
        mesh=mesh,
        compiler_params=pltpu.CompilerParams(use_tc_tiling_on_sc=False),
        scratch_types=[
            pltpu.VMEM((_KT, _T), jnp.int32),
            pltpu.VMEM((_CH * _T, H), jnp.float32),
            pltpu.VMEM((_NPT, H), jnp.float32),
            pltpu.VMEM_SHARED((N, H), jnp.float32),
            pltpu.SemaphoreType.DMA,
        ],
    )
    def scatter(msg_hbm, idx_hbm, zer_hbm, p_hbm, idx_v, msg_v, nbuf, acc_sh, sem):
        cid = lax.axis_index("c")
        sid = lax.axis_index("s")
        wid = sid * _NC + cid
        nrows = pl.ds(sid * _NPT, _NPT)
        pltpu.sync_copy(zer_hbm.at[nrows], nbuf)
        pltpu.sync_copy(nbuf, acc_sh.at[nrows])
        plsc.subcore_barrier()
        pltpu.sync_copy(idx_hbm.at[pl.ds(wid * _KT, _KT)], idx_v)
        for c in range(_NCHUNK):
            pltpu.sync_copy(
                msg_hbm.at[pl.ds(wid * _EPW + c * _CH * _T, _CH * _T)], msg_v)
            for b in range(_CH):
                pltpu.sync_copy(msg_v.at[pl.ds(b * _T, _T)],
                                acc_sh.at[idx_v.at[c * _CH + b]], add=True)
        plsc.subcore_barrier()
        pltpu.sync_copy(acc_sh.at[nrows], nbuf)
        pltpu.sync_copy(nbuf, p_hbm.at[cid, nrows])

    return scatter


def _final_body(p_ref, g_ref, bt_ref, w1_ref, b1_ref, w2_ref, b2_ref,
                y1_ref, y2_ref):
    p = p_ref[...]
    out = jnp.maximum(p[0] + p[1], 0.0)
    mean = jnp.mean(out, axis=0, keepdims=True)
    var = jnp.mean((out - mean) ** 2, axis=0, keepdims=True)
    yb = (out - mean) * lax.rsqrt(var + EPS) * g_ref[...] + bt_ref[...]
    z1 = jnp.dot(yb, w1_ref[...], preferred_element_type=jnp.float32) + b1_ref[...]
    z2 = jnp.dot(yb, w2_ref[...], preferred_element_type=jnp.float32) + b2_ref[...]
    y1_ref[...] = jax.nn.sigmoid(z1)
    y2_ref[...] = jax.nn.sigmoid(z2)


def _final(p, bn_gamma, bn_beta, yl_W, yl_b, yl2_W, yl2_b):
    return pl.pallas_call(
        _final_body,
        out_shape=(
            jax.ShapeDtypeStruct((N, 2), jnp.float32),
            jax.ShapeDtypeStruct((N, 1), jnp.float32),
        ),
        interpret=_INTERPRET,
    )(p, bn_gamma.reshape(1, H), bn_beta.reshape(1, H),
      yl_W, yl_b.reshape(1, 2), yl2_W, yl2_b.reshape(1, 1))


def kernel(n_feat, edge_index, e_feat, lin0_W, lin0_b, en1_W, en1_b,
           en2_W, en2_b, bn_gamma, bn_beta, yl_W, yl_b, yl2_W, yl2_b):
    src = edge_index[0].astype(jnp.int32)
    dst = edge_index[1].astype(jnp.int32)
    pad = jnp.zeros((_EPAD - E,), jnp.int32)
    src2d = jnp.concatenate([src, pad]).reshape(_EPAD // _T, _T)
    dst2d = jnp.concatenate([dst, pad]).reshape(_EPAD // _T, _T)
    zer = jnp.zeros((N, H), jnp.float32)

    out = _lin0(n_feat, lin0_W, lin0_b)
    w = _wnet(e_feat, en1_W, en1_b, en2_W, en2_b)

    gather1 = _build_gather1()
    gather2 = _build_gather2()
    scatter = _build_scatter()

    q, r = _make_qr()
    s = gather1(out, src2d)
    msg = _msg_single(s, w, q, r)
    p = scatter(msg, dst2d, zer)
    for _ in range(STEPS - 1):
        s0, s1 = gather2(p[0], p[1], src2d)
        msg = _msg_pair(s0, s1, w, q, r)
        p = scatter(msg, dst2d, zer)

    return _final(p, bn_gamma, bn_beta, yl_W, yl_b, yl2_W, yl2_b)


# speedup vs baseline: 3.1443x; 1.1776x over previous
import functools
import jax
import jax.numpy as jnp
from jax import lax
from jax.experimental import pallas as pl
from jax.experimental.pallas import tpu as pltpu
from jax.experimental.pallas import tpu_sc as plsc

N = 10000
E = 160000
DIN = 128
H = 32
ED = 16
EH = 128
STEPS = 3
EPS = 1e-5

_INTERPRET = False

_NC = 2
_NS = 16
_NW = _NC * _NS
_T = 128
_KT = 40
_EPW = _T * _KT
_EPAD = _NW * _EPW
_CH = 8
_NCHUNK = _KT // _CH
_NPT = N // _NS


def _lin0_body(x_ref, w_ref, b_ref, o_ref):
    y = jnp.dot(x_ref[...], w_ref[...], preferred_element_type=jnp.float32)
    o_ref[...] = jnp.maximum(y + b_ref[...], 0.0)


def _lin0(n_feat, lin0_W, lin0_b):
    return pl.pallas_call(
        _lin0_body,
        out_shape=jax.ShapeDtypeStruct((N, H), jnp.float32),
        interpret=_INTERPRET,
    )(n_feat, lin0_W, lin0_b.reshape(1, H))


_BE_W = 1280


def _ehnet_body(ef_ref, w1_ref, b1_ref, o_ref):
    eh = jnp.dot(ef_ref[...], w1_ref[...], preferred_element_type=jnp.float32)
    o_ref[...] = jnp.maximum(eh + b1_ref[...], 0.0)


def _ehnet(e_feat, en1_W, en1_b):
    grid = (E // _BE_W,)
    return pl.pallas_call(
        _ehnet_body,
        grid=grid,
        in_specs=[
            pl.BlockSpec((_BE_W, ED), lambda i: (i, 0)),
            pl.BlockSpec((ED, EH), lambda i: (0, 0)),
            pl.BlockSpec((1, EH), lambda i: (0, 0)),
        ],
        out_specs=pl.BlockSpec((_BE_W, EH), lambda i: (i, 0)),
        out_shape=jax.ShapeDtypeStruct((E, EH), jnp.float32),
        interpret=_INTERPRET,
    )(e_feat, en1_W, en1_b.reshape(1, EH))


_BE_M = 640
_NT_REAL = E // _BE_M
_NT_PAD = _EPAD // _BE_M


def _msg_compute(s, eh_ref, w2_ref, b2_ref, q_ref, r_ref, o_ref, i):
    wt = jnp.dot(eh_ref[...], w2_ref[...], preferred_element_type=jnp.float32)
    srep = jnp.dot(s, q_ref[...], preferred_element_type=jnp.float32)
    c = srep * wt
    cf = (c[:, 0:256] + c[:, 256:512]) + (c[:, 512:768] + c[:, 768:1024])
    m = (jnp.dot(cf, r_ref[...], preferred_element_type=jnp.float32)
         + jnp.dot(s, b2_ref[...], preferred_element_type=jnp.float32))
    o_ref[...] = m * jnp.where(i < _NT_REAL, 1.0, 0.0)


def _msg_body_single(s_ref, eh_ref, w2_ref, b2_ref, q_ref, r_ref, o_ref):
    _msg_compute(s_ref[...], eh_ref, w2_ref, b2_ref, q_ref, r_ref, o_ref,
                 pl.program_id(0))


def _msg_body_pair(s0_ref, s1_ref, eh_ref, w2_ref, b2_ref, q_ref, r_ref, o_ref):
    s = jnp.maximum(s0_ref[...] + s1_ref[...], 0.0)
    _msg_compute(s, eh_ref, w2_ref, b2_ref, q_ref, r_ref, o_ref,
                 pl.program_id(0))


def _make_qr():
    j = jnp.arange(H * H)
    q = (j[None, :] // H == jnp.arange(H)[:, None]).astype(jnp.float32)
    r = (jnp.arange(256)[:, None] % H == jnp.arange(H)[None, :]).astype(jnp.float32)
    return q, r


def _w_index(i):
    return (jnp.minimum(i, _NT_REAL - 1), 0)


_CONST_SPECS = [
    pl.BlockSpec((EH, H * H), lambda i: (0, 0)),
    pl.BlockSpec((H, H), lambda i: (0, 0)),
    pl.BlockSpec((H, H * H), lambda i: (0, 0)),
    pl.BlockSpec((256, H), lambda i: (0, 0)),
]


def _msg_single(s, ehb, w2b, b2m, q, r):
    return pl.pallas_call(
        _msg_body_single,
        grid=(_NT_PAD,),
        in_specs=[
            pl.BlockSpec((_BE_M, H), lambda i: (i, 0)),
            pl.BlockSpec((_BE_M, EH), _w_index),
        ] + _CONST_SPECS,
        out_specs=pl.BlockSpec((_BE_M, H), lambda i: (i, 0)),
        out_shape=jax.ShapeDtypeStruct((_EPAD, H), jnp.float32),
        interpret=_INTERPRET,
    )(s, ehb, w2b, b2m, q, r)


def _msg_pair(s0, s1, ehb, w2b, b2m, q, r):
    return pl.pallas_call(
        _msg_body_pair,
        grid=(_NT_PAD,),
        in_specs=[
            pl.BlockSpec((_BE_M, H), lambda i: (i, 0)),
            pl.BlockSpec((_BE_M, H), lambda i: (i, 0)),
            pl.BlockSpec((_BE_M, EH), _w_index),
        ] + _CONST_SPECS,
        out_specs=pl.BlockSpec((_BE_M, H), lambda i: (i, 0)),
        out_shape=jax.ShapeDtypeStruct((_EPAD, H), jnp.float32),
        interpret=_INTERPRET,
    )(s0, s1, ehb, w2b, b2m, q, r)


@functools.lru_cache(maxsize=None)
def _build_gather1():
    mesh = plsc.VectorSubcoreMesh(core_axis_name="c", subcore_axis_name="s")

    @functools.partial(
        pl.kernel,
        out_type=jax.ShapeDtypeStruct((_EPAD, H), jnp.float32),
        mesh=mesh,
        compiler_params=pltpu.CompilerParams(use_tc_tiling_on_sc=False),
        scratch_types=[
            pltpu.VMEM((_KT, _T), jnp.int32),
            pltpu.VMEM((_CH * _T, H), jnp.float32),
            pltpu.SemaphoreType.DMA,
        ],
    )
    def gather1(tab_hbm, idx_hbm, s_hbm, idx_v, rows_v, sem):
        wid = lax.axis_index("s") * _NC + lax.axis_index("c")
        pltpu.sync_copy(idx_hbm.at[pl.ds(wid * _KT, _KT)], idx_v)
        for c in range(_NCHUNK):
            cps = []
            for b in range(_CH):
                cps.append(pltpu.async_copy(
                    tab_hbm.at[idx_v.at[c * _CH + b]],
                    rows_v.at[pl.ds(b * _T, _T)], sem))
            for cp in cps:
                cp.wait()
            pltpu.sync_copy(
                rows_v, s_hbm.at[pl.ds(wid * _EPW + c * _CH * _T, _CH * _T)])

    return gather1


@functools.lru_cache(maxsize=None)
def _build_gather2():
    mesh = plsc.VectorSubcoreMesh(core_axis_name="c", subcore_axis_name="s")

    @functools.partial(
        pl.kernel,
        out_type=(jax.ShapeDtypeStruct((_EPAD, H), jnp.float32),
                  jax.ShapeDtypeStruct((_EPAD, H), jnp.float32)),
        mesh=mesh,
        compiler_params=pltpu.CompilerParams(use_tc_tiling_on_sc=False),
        scratch_types=[
            pltpu.VMEM((_KT, _T), jnp.int32),
            pltpu.VMEM((_CH * _T, H), jnp.float32),
            pltpu.VMEM((_CH * _T, H), jnp.float32),
            pltpu.SemaphoreType.DMA,
        ],
    )
    def gather2(t0_hbm, t1_hbm, idx_hbm, s0_hbm, s1_hbm, idx_v, r0_v, r1_v, sem):
        wid = lax.axis_index("s") * _NC + lax.axis_index("c")
        pltpu.sync_copy(idx_hbm.at[pl.ds(wid * _KT, _KT)], idx_v)
        for c in range(_NCHUNK):
            cps = []
            for b in range(_CH):
                iv = idx_v.at[c * _CH + b]
                cps.append(pltpu.async_copy(
                    t0_hbm.at[iv], r0_v.at[pl.ds(b * _T, _T)], sem))
                cps.append(pltpu.async_copy(
                    t1_hbm.at[iv], r1_v.at[pl.ds(b * _T, _T)], sem))
            for cp in cps:
                cp.wait()
            dst = pl.ds(wid * _EPW + c * _CH * _T, _CH * _T)
            pltpu.sync_copy(r0_v, s0_hbm.at[dst])
            pltpu.sync_copy(r1_v, s1_hbm.at[dst])

    return gather2


@functools.lru_cache(maxsize=None)
def _build_scatter():
    mesh = plsc.VectorSubcoreMesh(core_axis_name="c", subcore_axis_name="s")

    @functools.partial(
        pl.kernel,
        out_type=jax.ShapeDtypeStruct((_NC, N, H), jnp.float32),
        mesh=mesh,
        compiler_params=pltpu.CompilerParams(use_tc_tiling_on_sc=False),
        scratch_types=[
            pltpu.VMEM((_KT, _T), jnp.int32),
            pltpu.VMEM((_CH * _T, H), jnp.float32),
            pltpu.VMEM((_NPT, H), jnp.float32),
            pltpu.VMEM_SHARED((N, H), jnp.float32),
            pltpu.SemaphoreType.DMA,
        ],
    )
    def scatter(msg_hbm, idx_hbm, zer_hbm, p_hbm, idx_v, msg_v, nbuf, acc_sh, sem):
        cid = lax.axis_index("c")
        sid = lax.axis_index("s")
        wid = sid * _NC + cid
        nrows = pl.ds(sid * _NPT, _NPT)
        pltpu.sync_copy(zer_hbm.at[nrows], nbuf)
        pltpu.sync_copy(nbuf, acc_sh.at[nrows])
        plsc.subcore_barrier()
        pltpu.sync_copy(idx_hbm.at[pl.ds(wid * _KT, _KT)], idx_v)
        for c in range(_NCHUNK):
            pltpu.sync_copy(
                msg_hbm.at[pl.ds(wid * _EPW + c * _CH * _T, _CH * _T)], msg_v)
            for b in range(_CH):
                pltpu.sync_copy(msg_v.at[pl.ds(b * _T, _T)],
                                acc_sh.at[idx_v.at[c * _CH + b]], add=True)
        plsc.subcore_barrier()
        pltpu.sync_copy(acc_sh.at[nrows], nbuf)
        pltpu.sync_copy(nbuf, p_hbm.at[cid, nrows])

    return scatter


def _final_body(p_ref, g_ref, bt_ref, w1_ref, b1_ref, w2_ref, b2_ref,
                y1_ref, y2_ref):
    p = p_ref[...]
    out = jnp.maximum(p[0] + p[1], 0.0)
    mean = jnp.mean(out, axis=0, keepdims=True)
    var = jnp.mean((out - mean) ** 2, axis=0, keepdims=True)
    yb = (out - mean) * lax.rsqrt(var + EPS) * g_ref[...] + bt_ref[...]
    z1 = jnp.dot(yb, w1_ref[...], preferred_element_type=jnp.float32) + b1_ref[...]
    z2 = jnp.dot(yb, w2_ref[...], preferred_element_type=jnp.float32) + b2_ref[...]
    y1_ref[...] = jax.nn.sigmoid(z1)
    y2_ref[...] = jax.nn.sigmoid(z2)


def _final(p, bn_gamma, bn_beta, yl_W, yl_b, yl2_W, yl2_b):
    return pl.pallas_call(
        _final_body,
        out_shape=(
            jax.ShapeDtypeStruct((N, 2), jnp.float32),
            jax.ShapeDtypeStruct((N, 1), jnp.float32),
        ),
        interpret=_INTERPRET,
    )(p, bn_gamma.reshape(1, H), bn_beta.reshape(1, H),
      yl_W, yl_b.reshape(1, 2), yl2_W, yl2_b.reshape(1, 1))


def kernel(n_feat, edge_index, e_feat, lin0_W, lin0_b, en1_W, en1_b,
           en2_W, en2_b, bn_gamma, bn_beta, yl_W, yl_b, yl2_W, yl2_b):
    src = edge_index[0].astype(jnp.int32)
    dst = edge_index[1].astype(jnp.int32)
    pad = jnp.zeros((_EPAD - E,), jnp.int32)
    src2d = jnp.concatenate([src, pad]).reshape(_EPAD // _T, _T)
    dst2d = jnp.concatenate([dst, pad]).reshape(_EPAD // _T, _T)
    zer = jnp.zeros((N, H), jnp.float32)

    out = _lin0(n_feat, lin0_W, lin0_b)
    ehb = _ehnet(e_feat, en1_W, en1_b)
    w2b = en2_W
    b2m = en2_b.reshape(H, H)

    gather1 = _build_gather1()
    gather2 = _build_gather2()
    scatter = _build_scatter()

    q, r = _make_qr()
    s = gather1(out, src2d)
    msg = _msg_single(s, ehb, w2b, b2m, q, r)
    p = scatter(msg, dst2d, zer)
    for _ in range(STEPS - 1):
        s0, s1 = gather2(p[0], p[1], src2d)
        msg = _msg_pair(s0, s1, ehb, w2b, b2m, q, r)
        p = scatter(msg, dst2d, zer)

    return _final(p, bn_gamma, bn_beta, yl_W, yl_b, yl2_W, yl2_b)


# speedup vs baseline: 3.1684x; 1.0077x over previous
import functools
import jax
import jax.numpy as jnp
from jax import lax
from jax.experimental import pallas as pl
from jax.experimental.pallas import tpu as pltpu
from jax.experimental.pallas import tpu_sc as plsc

N = 10000
E = 160000
DIN = 128
H = 32
ED = 16
EH = 128
STEPS = 3
EPS = 1e-5

_INTERPRET = False

_NC = 2
_NS = 16
_NW = _NC * _NS
_T = 128
_KT = 40
_EPW = _T * _KT
_EPAD = _NW * _EPW
_CH = 8
_NCHUNK = _KT // _CH
_NPT = N // _NS


def _lin0_body(x_ref, w_ref, b_ref, o_ref):
    y = jnp.dot(x_ref[...], w_ref[...], preferred_element_type=jnp.float32)
    o_ref[...] = jnp.maximum(y + b_ref[...], 0.0)


def _lin0(n_feat, lin0_W, lin0_b):
    return pl.pallas_call(
        _lin0_body,
        out_shape=jax.ShapeDtypeStruct((N, H), jnp.float32),
        interpret=_INTERPRET,
    )(n_feat, lin0_W, lin0_b.reshape(1, H))


_BE_W = 1280


def _ehnet_body(ef_ref, w1_ref, b1_ref, o_ref):
    eh = jnp.dot(ef_ref[...], w1_ref[...], preferred_element_type=jnp.float32)
    o_ref[...] = jnp.maximum(eh + b1_ref[...], 0.0)


def _ehnet(e_feat, en1_W, en1_b):
    grid = (E // _BE_W,)
    return pl.pallas_call(
        _ehnet_body,
        grid=grid,
        in_specs=[
            pl.BlockSpec((_BE_W, ED), lambda i: (i, 0)),
            pl.BlockSpec((ED, EH), lambda i: (0, 0)),
            pl.BlockSpec((1, EH), lambda i: (0, 0)),
        ],
        out_specs=pl.BlockSpec((_BE_W, EH), lambda i: (i, 0)),
        out_shape=jax.ShapeDtypeStruct((E, EH), jnp.float32),
        interpret=_INTERPRET,
    )(e_feat, en1_W, en1_b.reshape(1, EH))


_BE_M = 640
_NT_REAL = E // _BE_M
_NT_PAD = _EPAD // _BE_M


def _msg_compute(s, eh_ref, w2_ref, b2_ref, q_ref, r_ref, o_ref, i):
    wt = jnp.dot(eh_ref[...], w2_ref[...], preferred_element_type=jnp.float32)
    srep = jnp.dot(s, q_ref[...], preferred_element_type=jnp.float32)
    c = srep * wt
    cf = (c[:, 0:256] + c[:, 256:512]) + (c[:, 512:768] + c[:, 768:1024])
    m = (jnp.dot(cf, r_ref[...], preferred_element_type=jnp.float32)
         + jnp.dot(s, b2_ref[...], preferred_element_type=jnp.float32))
    o_ref[...] = m * jnp.where(i < _NT_REAL, 1.0, 0.0)


def _msg_body_single(s_ref, eh_ref, w2_ref, b2_ref, q_ref, r_ref, o_ref):
    _msg_compute(s_ref[...], eh_ref, w2_ref, b2_ref, q_ref, r_ref, o_ref,
                 pl.program_id(0))


def _msg_body_pair(s0_ref, s1_ref, eh_ref, w2_ref, b2_ref, q_ref, r_ref, o_ref):
    s = jnp.maximum(s0_ref[...] + s1_ref[...], 0.0)
    _msg_compute(s, eh_ref, w2_ref, b2_ref, q_ref, r_ref, o_ref,
                 pl.program_id(0))


def _make_qr():
    j = jnp.arange(H * H)
    q = (j[None, :] // H == jnp.arange(H)[:, None]).astype(jnp.float32)
    r = (jnp.arange(256)[:, None] % H == jnp.arange(H)[None, :]).astype(jnp.float32)
    return q, r


def _w_index(i):
    return (jnp.minimum(i, _NT_REAL - 1), 0)


_CONST_SPECS = [
    pl.BlockSpec((EH, H * H), lambda i: (0, 0)),
    pl.BlockSpec((H, H), lambda i: (0, 0)),
    pl.BlockSpec((H, H * H), lambda i: (0, 0)),
    pl.BlockSpec((256, H), lambda i: (0, 0)),
]


def _msg_single(s, ehb, w2b, b2m, q, r):
    return pl.pallas_call(
        _msg_body_single,
        grid=(_NT_PAD,),
        in_specs=[
            pl.BlockSpec((_BE_M, H), lambda i: (i, 0)),
            pl.BlockSpec((_BE_M, EH), _w_index),
        ] + _CONST_SPECS,
        out_specs=pl.BlockSpec((_BE_M, H), lambda i: (i, 0)),
        out_shape=jax.ShapeDtypeStruct((_EPAD, H), jnp.float32),
        interpret=_INTERPRET,
    )(s, ehb, w2b, b2m, q, r)


def _msg_pair(s0, s1, ehb, w2b, b2m, q, r):
    return pl.pallas_call(
        _msg_body_pair,
        grid=(_NT_PAD,),
        in_specs=[
            pl.BlockSpec((_BE_M, H), lambda i: (i, 0)),
            pl.BlockSpec((_BE_M, H), lambda i: (i, 0)),
            pl.BlockSpec((_BE_M, EH), _w_index),
        ] + _CONST_SPECS,
        out_specs=pl.BlockSpec((_BE_M, H), lambda i: (i, 0)),
        out_shape=jax.ShapeDtypeStruct((_EPAD, H), jnp.float32),
        interpret=_INTERPRET,
    )(s0, s1, ehb, w2b, b2m, q, r)


@functools.lru_cache(maxsize=None)
def _build_gather1():
    mesh = plsc.VectorSubcoreMesh(core_axis_name="c", subcore_axis_name="s")

    @functools.partial(
        pl.kernel,
        out_type=jax.ShapeDtypeStruct((_EPAD, H), jnp.float32),
        mesh=mesh,
        compiler_params=pltpu.CompilerParams(use_tc_tiling_on_sc=False),
        scratch_types=[
            pltpu.VMEM((_KT, _T), jnp.int32),
            pltpu.VMEM((_CH * _T, H), jnp.float32),
            pltpu.SemaphoreType.DMA,
        ],
    )
    def gather1(tab_hbm, idx_hbm, s_hbm, idx_v, rows_v, sem):
        wid = lax.axis_index("s") * _NC + lax.axis_index("c")
        pltpu.sync_copy(idx_hbm.at[pl.ds(wid * _KT, _KT)], idx_v)
        for c in range(_NCHUNK):
            cps = []
            for b in range(_CH):
                cps.append(pltpu.async_copy(
                    tab_hbm.at[idx_v.at[c * _CH + b]],
                    rows_v.at[pl.ds(b * _T, _T)], sem))
            for cp in cps:
                cp.wait()
            pltpu.sync_copy(
                rows_v, s_hbm.at[pl.ds(wid * _EPW + c * _CH * _T, _CH * _T)])

    return gather1


@functools.lru_cache(maxsize=None)
def _build_gather2():
    mesh = plsc.VectorSubcoreMesh(core_axis_name="c", subcore_axis_name="s")

    @functools.partial(
        pl.kernel,
        out_type=(jax.ShapeDtypeStruct((_EPAD, H), jnp.float32),
                  jax.ShapeDtypeStruct((_EPAD, H), jnp.float32)),
        mesh=mesh,
        compiler_params=pltpu.CompilerParams(use_tc_tiling_on_sc=False),
        scratch_types=[
            pltpu.VMEM((_KT, _T), jnp.int32),
            pltpu.VMEM((_CH * _T, H), jnp.float32),
            pltpu.VMEM((_CH * _T, H), jnp.float32),
            pltpu.SemaphoreType.DMA,
        ],
    )
    def gather2(t0_hbm, t1_hbm, idx_hbm, s0_hbm, s1_hbm, idx_v, r0_v, r1_v, sem):
        wid = lax.axis_index("s") * _NC + lax.axis_index("c")
        pltpu.sync_copy(idx_hbm.at[pl.ds(wid * _KT, _KT)], idx_v)
        for c in range(_NCHUNK):
            cps = []
            for b in range(_CH):
                iv = idx_v.at[c * _CH + b]
                cps.append(pltpu.async_copy(
                    t0_hbm.at[iv], r0_v.at[pl.ds(b * _T, _T)], sem))
                cps.append(pltpu.async_copy(
                    t1_hbm.at[iv], r1_v.at[pl.ds(b * _T, _T)], sem))
            for cp in cps:
                cp.wait()
            dst = pl.ds(wid * _EPW + c * _CH * _T, _CH * _T)
            pltpu.sync_copy(r0_v, s0_hbm.at[dst])
            pltpu.sync_copy(r1_v, s1_hbm.at[dst])

    return gather2


@functools.lru_cache(maxsize=None)
def _build_scatter():
    mesh = plsc.VectorSubcoreMesh(core_axis_name="c", subcore_axis_name="s")

    @functools.partial(
        pl.kernel,
        out_type=jax.ShapeDtypeStruct((_NC, N, H), jnp.float32),
        mesh=mesh,
        compiler_params=pltpu.CompilerParams(use_tc_tiling_on_sc=False),
        scratch_types=[
            pltpu.VMEM((_KT, _T), jnp.int32),
            pltpu.VMEM((_CH * _T, H), jnp.float32),
            pltpu.VMEM((_CH * _T, H), jnp.float32),
            pltpu.VMEM((_NPT, H), jnp.float32),
            pltpu.VMEM_SHARED((N, H), jnp.float32),
            pltpu.SemaphoreType.DMA,
            pltpu.SemaphoreType.DMA,
        ],
    )
    def scatter(msg_hbm, idx_hbm, zer_hbm, p_hbm, idx_v, m0, m1, nbuf,
                acc_sh, lsem, ssem):
        cid = lax.axis_index("c")
        sid = lax.axis_index("s")
        wid = sid * _NC + cid
        nrows = pl.ds(sid * _NPT, _NPT)
        pltpu.sync_copy(zer_hbm.at[nrows], nbuf)
        pltpu.sync_copy(nbuf, acc_sh.at[nrows])
        plsc.subcore_barrier()
        pltpu.sync_copy(idx_hbm.at[pl.ds(wid * _KT, _KT)], idx_v)
        bufs = (m0, m1)
        loads = [None] * _NCHUNK
        scats = [[] for _ in range(_NCHUNK)]

        def start_load(c):
            loads[c] = pltpu.async_copy(
                msg_hbm.at[pl.ds(wid * _EPW + c * _CH * _T, _CH * _T)],
                bufs[c % 2], lsem)

        start_load(0)
        for c in range(_NCHUNK):
            if c + 1 < _NCHUNK:
                for cp in scats[c - 1] if c >= 1 else ():
                    cp.wait()
                start_load(c + 1)
            loads[c].wait()
            buf = bufs[c % 2]
            for b in range(_CH):
                scats[c].append(pltpu.async_copy(
                    buf.at[pl.ds(b * _T, _T)],
                    acc_sh.at[idx_v.at[c * _CH + b]], ssem, add=True))
        for cp in scats[_NCHUNK - 2] + scats[_NCHUNK - 1]:
            cp.wait()
        plsc.subcore_barrier()
        pltpu.sync_copy(acc_sh.at[nrows], nbuf)
        pltpu.sync_copy(nbuf, p_hbm.at[cid, nrows])

    return scatter


def _final_body(p_ref, g_ref, bt_ref, w1_ref, b1_ref, w2_ref, b2_ref,
                y1_ref, y2_ref):
    p = p_ref[...]
    out = jnp.maximum(p[0] + p[1], 0.0)
    mean = jnp.mean(out, axis=0, keepdims=True)
    var = jnp.mean((out - mean) ** 2, axis=0, keepdims=True)
    yb = (out - mean) * lax.rsqrt(var + EPS) * g_ref[...] + bt_ref[...]
    z1 = jnp.dot(yb, w1_ref[...], preferred_element_type=jnp.float32) + b1_ref[...]
    z2 = jnp.dot(yb, w2_ref[...], preferred_element_type=jnp.float32) + b2_ref[...]
    y1_ref[...] = jax.nn.sigmoid(z1)
    y2_ref[...] = jax.nn.sigmoid(z2)


def _final(p, bn_gamma, bn_beta, yl_W, yl_b, yl2_W, yl2_b):
    return pl.pallas_call(
        _final_body,
        out_shape=(
            jax.ShapeDtypeStruct((N, 2), jnp.float32),
            jax.ShapeDtypeStruct((N, 1), jnp.float32),
        ),
        interpret=_INTERPRET,
    )(p, bn_gamma.reshape(1, H), bn_beta.reshape(1, H),
      yl_W, yl_b.reshape(1, 2), yl2_W, yl2_b.reshape(1, 1))


def kernel(n_feat, edge_index, e_feat, lin0_W, lin0_b, en1_W, en1_b,
           en2_W, en2_b, bn_gamma, bn_beta, yl_W, yl_b, yl2_W, yl2_b):
    src = edge_index[0].astype(jnp.int32)
    dst = edge_index[1].astype(jnp.int32)
    pad = jnp.zeros((_EPAD - E,), jnp.int32)
    src2d = jnp.concatenate([src, pad]).reshape(_EPAD // _T, _T)
    dst2d = jnp.concatenate([dst, pad]).reshape(_EPAD // _T, _T)
    zer = jnp.zeros((N, H), jnp.float32)

    out = _lin0(n_feat, lin0_W, lin0_b)
    ehb = _ehnet(e_feat, en1_W, en1_b)
    w2b = en2_W
    b2m = en2_b.reshape(H, H)

    gather1 = _build_gather1()
    gather2 = _build_gather2()
    scatter = _build_scatter()

    q, r = _make_qr()
    s = gather1(out, src2d)
    msg = _msg_single(s, ehb, w2b, b2m, q, r)
    p = scatter(msg, dst2d, zer)
    for _ in range(STEPS - 1):
        s0, s1 = gather2(p[0], p[1], src2d)
        msg = _msg_pair(s0, s1, ehb, w2b, b2m, q, r)
        p = scatter(msg, dst2d, zer)

    return _final(p, bn_gamma, bn_beta, yl_W, yl_b, yl2_W, yl2_b)


# speedup vs baseline: 3.3730x; 1.0646x over previous
import functools
import jax
import jax.numpy as jnp
from jax import lax
from jax.experimental import pallas as pl
from jax.experimental.pallas import tpu as pltpu
from jax.experimental.pallas import tpu_sc as plsc

N = 10000
E = 160000
DIN = 128
H = 32
ED = 16
EH = 128
STEPS = 3
EPS = 1e-5

_INTERPRET = False

_NC = 2
_NS = 16
_NW = _NC * _NS
_T = 128
_KT = 40
_EPW = _T * _KT
_EPAD = _NW * _EPW
_CH = 8
_NCHUNK = _KT // _CH
_NPT = N // _NS


def _lin0_body(x_ref, w_ref, b_ref, o_ref):
    y = jnp.dot(x_ref[...], w_ref[...], preferred_element_type=jnp.float32)
    o_ref[...] = jnp.maximum(y + b_ref[...], 0.0)


def _lin0(n_feat, lin0_W, lin0_b):
    return pl.pallas_call(
        _lin0_body,
        out_shape=jax.ShapeDtypeStruct((N, H), jnp.float32),
        interpret=_INTERPRET,
    )(n_feat, lin0_W, lin0_b.reshape(1, H))


_BE_W = 1280


def _ehnet_body(ef_ref, w1_ref, b1_ref, o_ref):
    eh = jnp.dot(ef_ref[...], w1_ref[...], preferred_element_type=jnp.float32)
    o_ref[...] = jnp.maximum(eh + b1_ref[...], 0.0)


def _ehnet(e_feat, en1_W, en1_b):
    grid = (E // _BE_W,)
    return pl.pallas_call(
        _ehnet_body,
        grid=grid,
        in_specs=[
            pl.BlockSpec((_BE_W, ED), lambda i: (i, 0)),
            pl.BlockSpec((ED, EH), lambda i: (0, 0)),
            pl.BlockSpec((1, EH), lambda i: (0, 0)),
        ],
        out_specs=pl.BlockSpec((_BE_W, EH), lambda i: (i, 0)),
        out_shape=jax.ShapeDtypeStruct((E, EH), jnp.float32),
        interpret=_INTERPRET,
    )(e_feat, en1_W, en1_b.reshape(1, EH))


_BE_M = 640
_NT_REAL = E // _BE_M
_NT_PAD = _EPAD // _BE_M


def _msg_compute(s, eh_ref, w2_ref, b2_ref, q_ref, r_ref, o_ref, i):
    wt = jnp.dot(eh_ref[...], w2_ref[...], preferred_element_type=jnp.float32)
    srep = jnp.dot(s, q_ref[...], preferred_element_type=jnp.float32)
    c = srep * wt
    cf = (c[:, 0:256] + c[:, 256:512]) + (c[:, 512:768] + c[:, 768:1024])
    m = (jnp.dot(cf, r_ref[...], preferred_element_type=jnp.float32)
         + jnp.dot(s, b2_ref[...], preferred_element_type=jnp.float32))
    o_ref[...] = m * jnp.where(i < _NT_REAL, 1.0, 0.0)


def _msg_body_single(s_ref, eh_ref, w2_ref, b2_ref, q_ref, r_ref, o_ref):
    _msg_compute(s_ref[...], eh_ref, w2_ref, b2_ref, q_ref, r_ref, o_ref,
                 pl.program_id(0))


def _msg_body_pair(s0_ref, s1_ref, eh_ref, w2_ref, b2_ref, q_ref, r_ref, o_ref):
    s = jnp.maximum(s0_ref[...] + s1_ref[...], 0.0)
    _msg_compute(s, eh_ref, w2_ref, b2_ref, q_ref, r_ref, o_ref,
                 pl.program_id(0))


def _make_qr():
    j = jnp.arange(H * H)
    q = (j[None, :] // H == jnp.arange(H)[:, None]).astype(jnp.float32)
    r = (jnp.arange(256)[:, None] % H == jnp.arange(H)[None, :]).astype(jnp.float32)
    return q, r


def _w_index(i):
    return (jnp.minimum(i, _NT_REAL - 1), 0)


_CONST_SPECS = [
    pl.BlockSpec((EH, H * H), lambda i: (0, 0)),
    pl.BlockSpec((H, H), lambda i: (0, 0)),
    pl.BlockSpec((H, H * H), lambda i: (0, 0)),
    pl.BlockSpec((256, H), lambda i: (0, 0)),
]


def _msg_single(s, ehb, w2b, b2m, q, r):
    return pl.pallas_call(
        _msg_body_single,
        grid=(_NT_PAD,),
        in_specs=[
            pl.BlockSpec((_BE_M, H), lambda i: (i, 0)),
            pl.BlockSpec((_BE_M, EH), _w_index),
        ] + _CONST_SPECS,
        out_specs=pl.BlockSpec((_BE_M, H), lambda i: (i, 0)),
        out_shape=jax.ShapeDtypeStruct((_EPAD, H), jnp.float32),
        interpret=_INTERPRET,
    )(s, ehb, w2b, b2m, q, r)


def _msg_pair(s0, s1, ehb, w2b, b2m, q, r):
    return pl.pallas_call(
        _msg_body_pair,
        grid=(_NT_PAD,),
        in_specs=[
            pl.BlockSpec((_BE_M, H), lambda i: (i, 0)),
            pl.BlockSpec((_BE_M, H), lambda i: (i, 0)),
            pl.BlockSpec((_BE_M, EH), _w_index),
        ] + _CONST_SPECS,
        out_specs=pl.BlockSpec((_BE_M, H), lambda i: (i, 0)),
        out_shape=jax.ShapeDtypeStruct((_EPAD, H), jnp.float32),
        interpret=_INTERPRET,
    )(s0, s1, ehb, w2b, b2m, q, r)


@functools.lru_cache(maxsize=None)
def _build_gather1():
    mesh = plsc.VectorSubcoreMesh(core_axis_name="c", subcore_axis_name="s")

    @functools.partial(
        pl.kernel,
        out_type=jax.ShapeDtypeStruct((_EPAD, H), jnp.float32),
        mesh=mesh,
        compiler_params=pltpu.CompilerParams(use_tc_tiling_on_sc=False),
        scratch_types=[
            pltpu.VMEM((_KT, _T), jnp.int32),
            pltpu.VMEM((_CH * _T, H), jnp.float32),
            pltpu.SemaphoreType.DMA,
        ],
    )
    def gather1(tab_hbm, idx_hbm, s_hbm, idx_v, rows_v, sem):
        wid = lax.axis_index("s") * _NC + lax.axis_index("c")
        pltpu.sync_copy(idx_hbm.at[pl.ds(wid * _KT, _KT)], idx_v)
        for c in range(_NCHUNK):
            cps = []
            for b in range(_CH):
                cps.append(pltpu.async_copy(
                    tab_hbm.at[idx_v.at[c * _CH + b]],
                    rows_v.at[pl.ds(b * _T, _T)], sem))
            for cp in cps:
                cp.wait()
            pltpu.sync_copy(
                rows_v, s_hbm.at[pl.ds(wid * _EPW + c * _CH * _T, _CH * _T)])

    return gather1


@functools.lru_cache(maxsize=None)
def _build_gather2():
    mesh = plsc.VectorSubcoreMesh(core_axis_name="c", subcore_axis_name="s")

    @functools.partial(
        pl.kernel,
        out_type=(jax.ShapeDtypeStruct((_EPAD, H), jnp.float32),
                  jax.ShapeDtypeStruct((_EPAD, H), jnp.float32)),
        mesh=mesh,
        compiler_params=pltpu.CompilerParams(use_tc_tiling_on_sc=False),
        scratch_types=[
            pltpu.VMEM((_KT, _T), jnp.int32),
            pltpu.VMEM((_CH * _T, H), jnp.float32),
            pltpu.VMEM((_CH * _T, H), jnp.float32),
            pltpu.SemaphoreType.DMA,
        ],
    )
    def gather2(t0_hbm, t1_hbm, idx_hbm, s0_hbm, s1_hbm, idx_v, r0_v, r1_v, sem):
        wid = lax.axis_index("s") * _NC + lax.axis_index("c")
        pltpu.sync_copy(idx_hbm.at[pl.ds(wid * _KT, _KT)], idx_v)
        for c in range(_NCHUNK):
            cps = []
            for b in range(_CH):
                iv = idx_v.at[c * _CH + b]
                cps.append(pltpu.async_copy(
                    t0_hbm.at[iv], r0_v.at[pl.ds(b * _T, _T)], sem))
                cps.append(pltpu.async_copy(
                    t1_hbm.at[iv], r1_v.at[pl.ds(b * _T, _T)], sem))
            for cp in cps:
                cp.wait()
            dst = pl.ds(wid * _EPW + c * _CH * _T, _CH * _T)
            pltpu.sync_copy(r0_v, s0_hbm.at[dst])
            pltpu.sync_copy(r1_v, s1_hbm.at[dst])

    return gather2


@functools.lru_cache(maxsize=None)
def _build_scatter():
    mesh = plsc.VectorSubcoreMesh(core_axis_name="c", subcore_axis_name="s")

    @functools.partial(
        pl.kernel,
        out_type=(jax.ShapeDtypeStruct((N, H), jnp.float32),
                  jax.ShapeDtypeStruct((N, H), jnp.float32)),
        mesh=mesh,
        compiler_params=pltpu.CompilerParams(use_tc_tiling_on_sc=False),
        scratch_types=[
            pltpu.VMEM((_KT, _T), jnp.int32),
            pltpu.VMEM((_CH * _T, H), jnp.float32),
            pltpu.VMEM((_CH * _T, H), jnp.float32),
            pltpu.VMEM((_NPT, H), jnp.float32),
            pltpu.VMEM_SHARED((N, H), jnp.float32),
            pltpu.SemaphoreType.DMA,
            pltpu.SemaphoreType.DMA,
        ],
    )
    def scatter(msg_hbm, idx_hbm, zer_hbm, p0_hbm, p1_hbm, idx_v, m0, m1, nbuf,
                acc_sh, lsem, ssem):
        cid = lax.axis_index("c")
        sid = lax.axis_index("s")
        wid = sid * _NC + cid
        nrows = pl.ds(sid * _NPT, _NPT)
        pltpu.sync_copy(zer_hbm.at[nrows], nbuf)
        pltpu.sync_copy(nbuf, acc_sh.at[nrows])
        plsc.subcore_barrier()
        pltpu.sync_copy(idx_hbm.at[pl.ds(wid * _KT, _KT)], idx_v)
        bufs = (m0, m1)
        loads = [None] * _NCHUNK
        scats = [[] for _ in range(_NCHUNK)]

        def start_load(c):
            loads[c] = pltpu.async_copy(
                msg_hbm.at[pl.ds(wid * _EPW + c * _CH * _T, _CH * _T)],
                bufs[c % 2], lsem)

        start_load(0)
        for c in range(_NCHUNK):
            if c + 1 < _NCHUNK:
                for cp in scats[c - 1] if c >= 1 else ():
                    cp.wait()
                start_load(c + 1)
            loads[c].wait()
            buf = bufs[c % 2]
            for b in range(_CH):
                scats[c].append(pltpu.async_copy(
                    buf.at[pl.ds(b * _T, _T)],
                    acc_sh.at[idx_v.at[c * _CH + b]], ssem, add=True))
        for cp in scats[_NCHUNK - 2] + scats[_NCHUNK - 1]:
            cp.wait()
        plsc.subcore_barrier()
        pltpu.sync_copy(acc_sh.at[nrows], nbuf)

        @pl.when(cid == 0)
        def _():
            pltpu.sync_copy(nbuf, p0_hbm.at[nrows])

        @pl.when(cid == 1)
        def _():
            pltpu.sync_copy(nbuf, p1_hbm.at[nrows])

    return scatter


def _final_body(p0_ref, p1_ref, g_ref, bt_ref, w1_ref, b1_ref, w2_ref, b2_ref,
                y1_ref, y2_ref):
    out = jnp.maximum(p0_ref[...] + p1_ref[...], 0.0)
    mean = jnp.mean(out, axis=0, keepdims=True)
    var = jnp.mean((out - mean) ** 2, axis=0, keepdims=True)
    yb = (out - mean) * lax.rsqrt(var + EPS) * g_ref[...] + bt_ref[...]
    z1 = jnp.dot(yb, w1_ref[...], preferred_element_type=jnp.float32) + b1_ref[...]
    z2 = jnp.dot(yb, w2_ref[...], preferred_element_type=jnp.float32) + b2_ref[...]
    y1_ref[...] = jax.nn.sigmoid(z1)
    y2_ref[...] = jax.nn.sigmoid(z2)


def _final(p0, p1, bn_gamma, bn_beta, yl_W, yl_b, yl2_W, yl2_b):
    return pl.pallas_call(
        _final_body,
        out_shape=(
            jax.ShapeDtypeStruct((N, 2), jnp.float32),
            jax.ShapeDtypeStruct((N, 1), jnp.float32),
        ),
        interpret=_INTERPRET,
    )(p0, p1, bn_gamma.reshape(1, H), bn_beta.reshape(1, H),
      yl_W, yl_b.reshape(1, 2), yl2_W, yl2_b.reshape(1, 1))


def kernel(n_feat, edge_index, e_feat, lin0_W, lin0_b, en1_W, en1_b,
           en2_W, en2_b, bn_gamma, bn_beta, yl_W, yl_b, yl2_W, yl2_b):
    src = edge_index[0].astype(jnp.int32)
    dst = edge_index[1].astype(jnp.int32)
    pad = jnp.zeros((_EPAD - E,), jnp.int32)
    src2d = jnp.concatenate([src, pad]).reshape(_EPAD // _T, _T)
    dst2d = jnp.concatenate([dst, pad]).reshape(_EPAD // _T, _T)
    zer = jnp.zeros((N, H), jnp.float32)

    out = _lin0(n_feat, lin0_W, lin0_b)
    ehb = _ehnet(e_feat, en1_W, en1_b)
    w2b = en2_W
    b2m = en2_b.reshape(H, H)

    gather1 = _build_gather1()
    gather2 = _build_gather2()
    scatter = _build_scatter()

    q, r = _make_qr()
    s = gather1(out, src2d)
    msg = _msg_single(s, ehb, w2b, b2m, q, r)
    p0, p1 = scatter(msg, dst2d, zer)
    for _ in range(STEPS - 1):
        s0, s1 = gather2(p0, p1, src2d)
        msg = _msg_pair(s0, s1, ehb, w2b, b2m, q, r)
        p0, p1 = scatter(msg, dst2d, zer)

    return _final(p0, p1, bn_gamma, bn_beta, yl_W, yl_b, yl2_W, yl2_b)


# speedup vs baseline: 3.3998x; 1.0080x over previous
import functools
import jax
import jax.numpy as jnp
from jax import lax
from jax.experimental import pallas as pl
from jax.experimental.pallas import tpu as pltpu
from jax.experimental.pallas import tpu_sc as plsc

N = 10000
E = 160000
DIN = 128
H = 32
ED = 16
EH = 128
STEPS = 3
EPS = 1e-5

_INTERPRET = False

_NC = 2
_NS = 16
_NW = _NC * _NS
_T = 128
_KT = 40
_EPW = _T * _KT
_EPAD = _NW * _EPW
_CH = 8
_NCHUNK = _KT // _CH
_NPT = N // _NS


def _lin0_body(x_ref, w_ref, b_ref, o_ref):
    y = jnp.dot(x_ref[...], w_ref[...], preferred_element_type=jnp.float32)
    o_ref[...] = jnp.maximum(y + b_ref[...], 0.0)


def _lin0(n_feat, lin0_W, lin0_b):
    return pl.pallas_call(
        _lin0_body,
        out_shape=jax.ShapeDtypeStruct((N, H), jnp.float32),
        interpret=_INTERPRET,
    )(n_feat, lin0_W, lin0_b.reshape(1, H))


_BE_W = 1280


def _ehnet_body(ef_ref, w1_ref, b1_ref, o_ref):
    eh = jnp.dot(ef_ref[...], w1_ref[...], preferred_element_type=jnp.float32)
    o_ref[...] = jnp.maximum(eh + b1_ref[...], 0.0)


def _ehnet(e_feat, en1_W, en1_b):
    grid = (E // _BE_W,)
    return pl.pallas_call(
        _ehnet_body,
        grid=grid,
        in_specs=[
            pl.BlockSpec((_BE_W, ED), lambda i: (i, 0)),
            pl.BlockSpec((ED, EH), lambda i: (0, 0)),
            pl.BlockSpec((1, EH), lambda i: (0, 0)),
        ],
        out_specs=pl.BlockSpec((_BE_W, EH), lambda i: (i, 0)),
        out_shape=jax.ShapeDtypeStruct((E, EH), jnp.float32),
        interpret=_INTERPRET,
    )(e_feat, en1_W, en1_b.reshape(1, EH))


_BE_M = 640
_NT_REAL = E // _BE_M
_NT_PAD = _EPAD // _BE_M


def _msg_compute(s, eh_ref, w2_ref, b2_ref, q_ref, r_ref, o_ref, i):
    wt = jnp.dot(eh_ref[...].astype(jnp.bfloat16), w2_ref[...].astype(jnp.bfloat16),
                 preferred_element_type=jnp.float32)
    srep = jnp.dot(s, q_ref[...], preferred_element_type=jnp.float32)
    c = srep * wt
    cf = (c[:, 0:256] + c[:, 256:512]) + (c[:, 512:768] + c[:, 768:1024])
    m = (jnp.dot(cf, r_ref[...], preferred_element_type=jnp.float32)
         + jnp.dot(s, b2_ref[...], preferred_element_type=jnp.float32))
    o_ref[...] = m * jnp.where(i < _NT_REAL, 1.0, 0.0)


def _make_qr():
    j = jnp.arange(H * H)
    q = (j[None, :] // H == jnp.arange(H)[:, None]).astype(jnp.float32)
    r = (jnp.arange(256)[:, None] % H == jnp.arange(H)[None, :]).astype(jnp.float32)
    return q, r


def _w_index(i):
    return (jnp.minimum(i, _NT_REAL - 1), 0)


_CONST_SPECS = [
    pl.BlockSpec((EH, H * H), lambda i: (0, 0)),
    pl.BlockSpec((H, H), lambda i: (0, 0)),
    pl.BlockSpec((H, H * H), lambda i: (0, 0)),
    pl.BlockSpec((256, H), lambda i: (0, 0)),
]


def _msg_half(s_list, ehb, w2b, b2m, q, r, base, ntiles, nreal):
    def s_index(i):
        return (base + i, 0)

    def w_index(i):
        return (jnp.minimum(base + i, E // _BE_M - 1), 0)

    def body(*refs):
        i = pl.program_id(0)
        if len(refs) == 7:
            s_ref, eh_ref, w2_ref, b2_ref, q_ref, r_ref, o_ref = refs
            s = s_ref[...]
        else:
            s0_ref, s1_ref, eh_ref, w2_ref, b2_ref, q_ref, r_ref, o_ref = refs
            s = jnp.maximum(s0_ref[...] + s1_ref[...], 0.0)
        wt = jnp.dot(eh_ref[...], w2_ref[...], preferred_element_type=jnp.float32)
        srep = jnp.dot(s, q_ref[...], preferred_element_type=jnp.float32)
        c = srep * wt
        cf = (c[:, 0:256] + c[:, 256:512]) + (c[:, 512:768] + c[:, 768:1024])
        m = (jnp.dot(cf, r_ref[...], preferred_element_type=jnp.float32)
             + jnp.dot(s, b2_ref[...], preferred_element_type=jnp.float32))
        o_ref[...] = m * jnp.where(i < nreal, 1.0, 0.0)

    s_specs = [pl.BlockSpec((_BE_M, H), s_index) for _ in s_list]
    return pl.pallas_call(
        body,
        grid=(ntiles,),
        in_specs=s_specs + [pl.BlockSpec((_BE_M, EH), w_index)] + _CONST_SPECS,
        out_specs=pl.BlockSpec((_BE_M, H), lambda i: (i, 0)),
        out_shape=jax.ShapeDtypeStruct((ntiles * _BE_M, H), jnp.float32),
        interpret=_INTERPRET,
    )(*s_list, ehb, w2b, b2m, q, r)


@functools.lru_cache(maxsize=None)
def _build_gather1():
    mesh = plsc.VectorSubcoreMesh(core_axis_name="c", subcore_axis_name="s")

    @functools.partial(
        pl.kernel,
        out_type=jax.ShapeDtypeStruct((_EPAD, H), jnp.float32),
        mesh=mesh,
        compiler_params=pltpu.CompilerParams(use_tc_tiling_on_sc=False),
        scratch_types=[
            pltpu.VMEM((_KT, _T), jnp.int32),
            pltpu.VMEM((_CH * _T, H), jnp.float32),
            pltpu.SemaphoreType.DMA,
        ],
    )
    def gather1(tab_hbm, idx_hbm, s_hbm, idx_v, rows_v, sem):
        wid = lax.axis_index("s") * _NC + lax.axis_index("c")
        pltpu.sync_copy(idx_hbm.at[pl.ds(wid * _KT, _KT)], idx_v)
        for c in range(_NCHUNK):
            cps = []
            for b in range(_CH):
                cps.append(pltpu.async_copy(
                    tab_hbm.at[idx_v.at[c * _CH + b]],
                    rows_v.at[pl.ds(b * _T, _T)], sem))
            for cp in cps:
                cp.wait()
            pltpu.sync_copy(
                rows_v, s_hbm.at[pl.ds(wid * _EPW + c * _CH * _T, _CH * _T)])

    return gather1


@functools.lru_cache(maxsize=None)
def _build_gather2():
    mesh = plsc.VectorSubcoreMesh(core_axis_name="c", subcore_axis_name="s")

    @functools.partial(
        pl.kernel,
        out_type=(jax.ShapeDtypeStruct((_EPAD, H), jnp.float32),
                  jax.ShapeDtypeStruct((_EPAD, H), jnp.float32)),
        mesh=mesh,
        compiler_params=pltpu.CompilerParams(use_tc_tiling_on_sc=False),
        scratch_types=[
            pltpu.VMEM((_KT, _T), jnp.int32),
            pltpu.VMEM((_CH * _T, H), jnp.float32),
            pltpu.VMEM((_CH * _T, H), jnp.float32),
            pltpu.SemaphoreType.DMA,
        ],
    )
    def gather2(t0_hbm, t1_hbm, idx_hbm, s0_hbm, s1_hbm, idx_v, r0_v, r1_v, sem):
        wid = lax.axis_index("s") * _NC + lax.axis_index("c")
        pltpu.sync_copy(idx_hbm.at[pl.ds(wid * _KT, _KT)], idx_v)
        for c in range(_NCHUNK):
            cps = []
            for b in range(_CH):
                iv = idx_v.at[c * _CH + b]
                cps.append(pltpu.async_copy(
                    t0_hbm.at[iv], r0_v.at[pl.ds(b * _T, _T)], sem))
                cps.append(pltpu.async_copy(
                    t1_hbm.at[iv], r1_v.at[pl.ds(b * _T, _T)], sem))
            for cp in cps:
                cp.wait()
            dst = pl.ds(wid * _EPW + c * _CH * _T, _CH * _T)
            pltpu.sync_copy(r0_v, s0_hbm.at[dst])
            pltpu.sync_copy(r1_v, s1_hbm.at[dst])

    return gather2


@functools.lru_cache(maxsize=None)
def _build_scatter(kt, ch):
    nchunk = kt // ch
    epw = kt * _T
    mesh = plsc.VectorSubcoreMesh(core_axis_name="c", subcore_axis_name="s")

    @functools.partial(
        pl.kernel,
        out_type=(jax.ShapeDtypeStruct((N, H), jnp.float32),
                  jax.ShapeDtypeStruct((N, H), jnp.float32)),
        mesh=mesh,
        compiler_params=pltpu.CompilerParams(use_tc_tiling_on_sc=False),
        scratch_types=[
            pltpu.VMEM((kt, _T), jnp.int32),
            pltpu.VMEM((ch * _T, H), jnp.float32),
            pltpu.VMEM((ch * _T, H), jnp.float32),
            pltpu.VMEM((_NPT, H), jnp.float32),
            pltpu.VMEM_SHARED((N, H), jnp.float32),
            pltpu.SemaphoreType.DMA,
            pltpu.SemaphoreType.DMA,
        ],
    )
    def scatter(msg_hbm, idx_hbm, i0_hbm, i1_hbm, p0_hbm, p1_hbm, idx_v, m0,
                m1, nbuf, acc_sh, lsem, ssem):
        cid = lax.axis_index("c")
        sid = lax.axis_index("s")
        wid = sid * _NC + cid
        nrows = pl.ds(sid * _NPT, _NPT)

        @pl.when(cid == 0)
        def _():
            pltpu.sync_copy(i0_hbm.at[nrows], nbuf)

        @pl.when(cid == 1)
        def _():
            pltpu.sync_copy(i1_hbm.at[nrows], nbuf)

        pltpu.sync_copy(nbuf, acc_sh.at[nrows])
        plsc.subcore_barrier()
        pltpu.sync_copy(idx_hbm.at[pl.ds(wid * kt, kt)], idx_v)
        bufs = (m0, m1)
        loads = [None] * nchunk
        scats = [[] for _ in range(nchunk)]

        def start_load(c):
            loads[c] = pltpu.async_copy(
                msg_hbm.at[pl.ds(wid * epw + c * ch * _T, ch * _T)],
                bufs[c % 2], lsem)

        start_load(0)
        for c in range(nchunk):
            if c + 1 < nchunk:
                for cp in scats[c - 1] if c >= 1 else ():
                    cp.wait()
                start_load(c + 1)
            loads[c].wait()
            buf = bufs[c % 2]
            for b in range(ch):
                scats[c].append(pltpu.async_copy(
                    buf.at[pl.ds(b * _T, _T)],
                    acc_sh.at[idx_v.at[c * ch + b]], ssem, add=True))
        for cp in (scats[nchunk - 2] if nchunk >= 2 else []) + scats[nchunk - 1]:
            cp.wait()
        plsc.subcore_barrier()
        pltpu.sync_copy(acc_sh.at[nrows], nbuf)

        @pl.when(cid == 0)
        def _():
            pltpu.sync_copy(nbuf, p0_hbm.at[nrows])

        @pl.when(cid == 1)
        def _():
            pltpu.sync_copy(nbuf, p1_hbm.at[nrows])

    return scatter


def _final_body(p0_ref, p1_ref, g_ref, bt_ref, w1_ref, b1_ref, w2_ref, b2_ref,
                y1_ref, y2_ref):
    out = jnp.maximum(p0_ref[...] + p1_ref[...], 0.0)
    mean = jnp.mean(out, axis=0, keepdims=True)
    var = jnp.mean((out - mean) ** 2, axis=0, keepdims=True)
    yb = (out - mean) * lax.rsqrt(var + EPS) * g_ref[...] + bt_ref[...]
    z1 = jnp.dot(yb, w1_ref[...], preferred_element_type=jnp.float32) + b1_ref[...]
    z2 = jnp.dot(yb, w2_ref[...], preferred_element_type=jnp.float32) + b2_ref[...]
    y1_ref[...] = jax.nn.sigmoid(z1)
    y2_ref[...] = jax.nn.sigmoid(z2)


def _final(p0, p1, bn_gamma, bn_beta, yl_W, yl_b, yl2_W, yl2_b):
    return pl.pallas_call(
        _final_body,
        out_shape=(
            jax.ShapeDtypeStruct((N, 2), jnp.float32),
            jax.ShapeDtypeStruct((N, 1), jnp.float32),
        ),
        interpret=_INTERPRET,
    )(p0, p1, bn_gamma.reshape(1, H), bn_beta.reshape(1, H),
      yl_W, yl_b.reshape(1, 2), yl2_W, yl2_b.reshape(1, 1))


def kernel(n_feat, edge_index, e_feat, lin0_W, lin0_b, en1_W, en1_b,
           en2_W, en2_b, bn_gamma, bn_beta, yl_W, yl_b, yl2_W, yl2_b):
    src = edge_index[0].astype(jnp.int32)
    dst = edge_index[1].astype(jnp.int32)
    pad = jnp.zeros((_EPAD - E,), jnp.int32)
    src2d = jnp.concatenate([src, pad]).reshape(_EPAD // _T, _T)
    dst2d = jnp.concatenate([dst, pad]).reshape(_EPAD // _T, _T)
    zer = jnp.zeros((N, H), jnp.float32)

    out = _lin0(n_feat, lin0_W, lin0_b)
    ehb = _ehnet(e_feat, en1_W, en1_b)
    w2b = en2_W
    b2m = en2_b.reshape(H, H)

    gather1 = _build_gather1()
    gather2 = _build_gather2()
    scatter_h = _build_scatter(_KT // 2, 10)
    q, r = _make_qr()

    nt_half = _NT_PAD // 2
    nreal_b = E // _BE_M - nt_half
    dst_a, dst_b = dst2d[:_EPAD // _T // 2], dst2d[_EPAD // _T // 2:]

    def step(s_list):
        msg_a = _msg_half(s_list, ehb, w2b, b2m, q, r, 0, nt_half, nt_half)
        msg_b = _msg_half(s_list, ehb, w2b, b2m, q, r, nt_half, nt_half, nreal_b)
        q0, q1 = scatter_h(msg_a, dst_a, zer, zer)
        return scatter_h(msg_b, dst_b, q0, q1)

    s = gather1(out, src2d)
    p0, p1 = step([s])
    for _ in range(STEPS - 1):
        s0, s1 = gather2(p0, p1, src2d)
        p0, p1 = step([s0, s1])

    return _final(p0, p1, bn_gamma, bn_beta, yl_W, yl_b, yl2_W, yl2_b)
